# baseline (speedup 1.0000x reference)
import functools

import jax
import jax.numpy as jnp
from jax import lax
from jax.experimental import pallas as pl
from jax.experimental.pallas import tpu as pltpu
from jax.experimental.pallas import tpu_sc as plsc

_K = 64
_INF = float('inf')

_NC, _NS = 2, 16
_NW = _NC * _NS
_CHUNK = 128
_NBUF = 4


def _dot(a, b, prec=None):
    return lax.dot_general(a, b, (((1,), (0,)), ((), ())),
                           precision=prec, preferred_element_type=jnp.float32)


def _split3(v):
    hi = v.astype(jnp.bfloat16)
    r1 = v - hi.astype(jnp.float32)
    mid = r1.astype(jnp.bfloat16)
    lo = (r1 - mid.astype(jnp.float32)).astype(jnp.bfloat16)
    return hi, mid, lo


def _gdot(oh, triple, data_left=False):
    ohb = oh.astype(jnp.bfloat16)
    if data_left:
        w = triple[0].shape[0]
        r = _dot(jnp.concatenate(triple, axis=0), ohb)
        return (r[:w] + r[w:2 * w]) + r[2 * w:]
    w = triple[0].shape[1]
    r = _dot(ohb, jnp.concatenate(triple, axis=1))
    return (r[:, :w] + r[:, w:2 * w]) + r[:, 2 * w:]


def _sqdist(q, pt):
    d0 = (q[:, 0:1] - pt[0:1, :]) ** 2
    d1 = (q[:, 1:2] - pt[1:2, :]) ** 2
    d2 = (q[:, 2:3] - pt[2:3, :]) ** 2
    return (d0 + d1) + d2


def _knn_select(D, d_ref):
    M, P = D.shape
    d_ref[...] = D
    iot = lax.broadcasted_iota(jnp.int32, (M, P), 1)
    kio = lax.broadcasted_iota(jnp.int32, (M, _K), 1)
    big = jnp.int32(2 ** 30)

    def step(k, carry):
        nidx, mp, cp = carry
        D = d_ref[...]
        live = (D > mp) | ((D == mp) & (iot > cp))
        m = jnp.min(jnp.where(live, D, _INF), axis=1, keepdims=True)
        col = jnp.min(jnp.where(live & (D == m), iot, big),
                      axis=1, keepdims=True)
        return jnp.where(kio == k, col, nidx), m, col

    carry0 = (jnp.zeros((M, _K), jnp.int32),
              jnp.full((M, 1), -_INF, jnp.float32),
              jnp.full((M, 1), -1, jnp.int32))
    return lax.fori_loop(0, _K, step, carry0)[0]


def _sa1_sel_body(ap_ref, apt_ref, i0c_ref, i0r_ref, i1c_ref, i1r_ref,
                  a1t_ref, q1w_ref, b1v_ref,
                  nidx_ref, p1e_ref, q1e_ref, pos1_ref, pos1t_ref, d_ref):
    ap = ap_ref[0]
    apt = apt_ref[0]
    i0c = i0c_ref[0]
    i0r = i0r_ref[0]
    i1c = i1c_ref[0]
    i1r = i1r_ref[0]
    N, M0, M1 = ap.shape[0], i0c.shape[0], i1c.shape[0]

    CH = min(N, 1024)
    ap3 = _split3(ap)
    apt3 = _split3(apt)
    pos0 = jnp.zeros((M0, 3), jnp.float32)
    pos0t = jnp.zeros((3, M0), jnp.float32)
    for s in range(0, N, CH):
        ii = lax.broadcasted_iota(jnp.int32, (M0, CH), 1) + s
        pos0 = pos0 + _gdot(i0c == ii, tuple(t[s:s + CH] for t in ap3))
        jj = lax.broadcasted_iota(jnp.int32, (CH, M0), 0) + s
        pos0t = pos0t + _gdot(jj == i0r, tuple(t[:, s:s + CH] for t in apt3),
                              data_left=True)

    oh1 = i1c == lax.broadcasted_iota(jnp.int32, (M1, M0), 1)
    q1 = _gdot(oh1, _split3(pos0))
    oh1t = lax.broadcasted_iota(jnp.int32, (M0, M1), 0) == i1r
    q1t = _gdot(oh1t, _split3(pos0t), data_left=True)

    p1e_ref[0] = _dot(pos0, a1t_ref[...], prec=None)
    q1e_ref[0] = _dot(q1, q1w_ref[...], prec=None) + b1v_ref[...]
    nidx_ref[0] = _knn_select(_sqdist(q1, pos0t), d_ref)
    pos1_ref[0] = q1
    pos1t_ref[0] = q1t


def _sa2_sel_body(p1_ref, p1t_ref, f1_ref, i2c_ref,
                  wp2t_ref, wf2t_ref, q2w_ref, b1v_ref,
                  nidx_ref, p2e_ref, q2e_ref, pos2_ref, d_ref):
    pos1 = p1_ref[0]
    pos1t = p1t_ref[0]
    feat1 = f1_ref[0]
    i2c = i2c_ref[0]
    M1, M2 = pos1.shape[0], i2c.shape[0]

    oh2 = i2c == lax.broadcasted_iota(jnp.int32, (M2, M1), 1)
    q2 = _gdot(oh2, _split3(pos1))
    p2e_ref[0] = (_dot(feat1, wf2t_ref[...], prec=None) +
                  _dot(pos1, wp2t_ref[...], prec=None))
    q2e_ref[0] = _dot(q2, q2w_ref[...], prec=None) + b1v_ref[...]
    nidx_ref[0] = _knn_select(_sqdist(q2, pos1t), d_ref)
    pos2_ref[0] = q2


def _mlp_max_body(h_ref, qe_ref, w2t_ref, b2v_ref, w3t_ref, b3v_ref, out_ref,
                  *, kchunk):
    KM, C1 = h_ref.shape
    M = qe_ref.shape[1]
    Kc = min(kchunk, KM // M)
    nch = (KM // M) // Kc
    qe = qe_ref[0]
    W2t, b2 = w2t_ref[...], b2v_ref[...]
    W3t, b3 = w3t_ref[...], b3v_ref[...]
    acc = jnp.full((M, W3t.shape[1]), -_INF, jnp.float32)
    for c in range(nch):
        blk = h_ref[pl.ds(c * Kc * M, Kc * M), :].reshape(Kc, M, C1)
        x = jnp.maximum(blk + qe, 0.0).reshape(Kc * M, C1)
        x = jnp.maximum(_dot(x, W2t, prec=None) + b2, 0.0)
        x = jnp.maximum(_dot(x, W3t, prec=None) + b3, 0.0)
        acc = jnp.maximum(acc, jnp.max(x.reshape(Kc, M, W3t.shape[1]), axis=0))
    out_ref[0] = acc


def _sc_gather(table, idx3d, C):
    cpw = idx3d.shape[1]
    rows_out = _NW * cpw * _CHUNK
    ngroups = cpw // _NBUF
    mesh = plsc.VectorSubcoreMesh(core_axis_name="c", subcore_axis_name="s")

    @functools.partial(
        pl.kernel, mesh=mesh,
        out_type=jax.ShapeDtypeStruct((rows_out, C), jnp.float32),
        scratch_types=[
            pltpu.VMEM((cpw, _CHUNK), jnp.int32),
            pltpu.VMEM((_NBUF, _CHUNK, C), jnp.float32),
            pltpu.SemaphoreType.DMA,
            pltpu.SemaphoreType.DMA,
        ],
    )
    def gk(table_hbm, idx_hbm, out_hbm, idx_v, rows_v, gsem, wsem):
        wid = lax.axis_index("s") * _NC + lax.axis_index("c")
        base = wid * cpw * _CHUNK
        pltpu.sync_copy(idx_hbm.at[wid], idx_v)

        def group(g, carry):
            row0 = base + g * (_NBUF * _CHUNK)
            gd, wd = [], []
            for j in range(_NBUF):
                gd.append(pltpu.async_copy(
                    table_hbm.at[idx_v.at[g * _NBUF + j]], rows_v.at[j], gsem))
            for j in range(_NBUF):
                gd[j].wait()
            for j in range(_NBUF):
                wd.append(pltpu.async_copy(
                    rows_v.at[j],
                    out_hbm.at[pl.ds(row0 + j * _CHUNK, _CHUNK)], wsem))
            for j in range(_NBUF):
                wd[j].wait()
            return carry

        lax.fori_loop(0, ngroups, group, 0)

    return gk(table, idx3d)


def _fin_body(p2_ref, f2_ref, fp1_ref, ff1_ref, fb1_ref, f2w_ref, fb2_ref,
              f3w_ref, fb3_ref, c1w_ref, c1b_ref, c2w_ref, c2b_ref,
              c3w_ref, c3b_ref, out_ref):
    pos2 = p2_ref[0]
    feat2 = f2_ref[0]
    x = _dot(pos2, fp1_ref[...], prec=None) + _dot(feat2, ff1_ref[...], prec=None)
    x = jnp.maximum(x + fb1_ref[...], 0.0)
    x = jnp.maximum(_dot(x, f2w_ref[...], prec=None) + fb2_ref[...], 0.0)
    x = jnp.maximum(_dot(x, f3w_ref[...], prec=None) + fb3_ref[...], 0.0)
    pooled = jnp.max(x, axis=0, keepdims=True)
    y = jnp.maximum(_dot(pooled, c1w_ref[...], prec=None) + c1b_ref[...], 0.0)
    y = jnp.maximum(_dot(y, c2w_ref[...], prec=None) + c2b_ref[...], 0.0)
    out_ref[0] = _dot(y, c3w_ref[...], prec=None) + c3b_ref[...]


def _full(shape):
    return pl.BlockSpec(shape, lambda b: (0,) * len(shape))


def _batched(shape):
    return pl.BlockSpec((1,) + shape, lambda b: (b,) + (0,) * len(shape))


def _flat_gather_idx(nidx, src_rows):
    B = nidx.shape[0]
    off = (jnp.arange(B, dtype=jnp.int32) * src_rows)[:, None, None]
    flat = (jnp.transpose(nidx, (0, 2, 1)) + off).reshape(-1)
    return flat.reshape(_NW, -1, _CHUNK)


def kernel(all_points, idx0, idx1, idx2, sa1_params, sa2_params, fin_params, fc_params):
    B, N, _ = all_points.shape
    M0, M1, M2 = idx0.shape[1], idx1.shape[1], idx2.shape[1]
    f32 = jnp.float32

    ap = all_points.astype(f32)
    apt = jnp.transpose(ap, (0, 2, 1))
    i0 = idx0.astype(jnp.int32)
    i1 = idx1.astype(jnp.int32)
    i2 = idx2.astype(jnp.int32)
    i0c, i0r = i0[:, :, None], i0[:, None, :]
    i1c, i1r = i1[:, :, None], i1[:, None, :]
    i2c = i2[:, :, None]

    (W1, g1, b1), (W2, g2, b2), (W3, g3, b3) = sa1_params
    a1t = ((W1[:, :3] + W1[:, 3:]) * g1[:, None]).T
    q1w = (-(W1[:, :3]) * g1[:, None]).T
    s1 = dict(a1t=a1t, q1w=q1w, b1v=b1[None, :],
              w2t=(W2 * g2[:, None]).T, b2v=b2[None, :],
              w3t=(W3 * g3[:, None]).T, b3v=b3[None, :])

    (V1, h1, c1), (V2, h2, c2), (V3, h3, c3) = sa2_params
    wp2t = (V1[:, :3] * h1[:, None]).T
    wf2t = (V1[:, 3:] * h1[:, None]).T
    s2 = dict(wp2t=wp2t, wf2t=wf2t, q2w=-wp2t, b1v=c1[None, :],
              w2t=(V2 * h2[:, None]).T, b2v=c2[None, :],
              w3t=(V3 * h3[:, None]).T, b3v=c3[None, :])

    (U1, e1, d1), (U2, e2, d2), (U3, e3, d3) = fin_params
    fin = dict(fp1=(U1[:, :3] * e1[:, None]).T, ff1=(U1[:, 3:] * e1[:, None]).T,
               fb1=d1[None, :],
               f2w=(U2 * e2[:, None]).T, fb2=d2[None, :],
               f3w=(U3 * e3[:, None]).T, fb3=d3[None, :])
    F1, fg1, fb1, F2, fg2, fb2, F3, fb3 = fc_params
    fc = dict(c1w=(F1 * fg1[:, None]).T, c1b=fb1[None, :],
              c2w=(F2 * fg2[:, None]).T, c2b=fb2[None, :],
              c3w=F3.T, c3b=fb3[None, :])

    C1h = s1['w2t'].shape[0]
    C1 = s1['w3t'].shape[1]
    C2h = s2['w2t'].shape[0]
    C2 = s2['w3t'].shape[1]

    s1_keys = ['a1t', 'q1w', 'b1v']
    nidx1, p1e, q1e, pos1, pos1t = pl.pallas_call(
        _sa1_sel_body,
        grid=(B,),
        in_specs=[_batched((N, 3)), _batched((3, N)),
                  _batched((M0, 1)), _batched((1, M0)),
                  _batched((M1, 1)), _batched((1, M1))] +
                 [_full(s1[k].shape) for k in s1_keys],
        out_specs=[_batched((M1, _K)), _batched((M0, C1h)),
                   _batched((M1, C1h)), _batched((M1, 3)), _batched((3, M1))],
        out_shape=[jax.ShapeDtypeStruct((B, M1, _K), jnp.int32),
                   jax.ShapeDtypeStruct((B, M0, C1h), f32),
                   jax.ShapeDtypeStruct((B, M1, C1h), f32),
                   jax.ShapeDtypeStruct((B, M1, 3), f32),
                   jax.ShapeDtypeStruct((B, 3, M1), f32)],
        scratch_shapes=[pltpu.VMEM((M1, M0), f32)],
    )(ap, apt, i0c, i0r, i1c, i1r, *[s1[k] for k in s1_keys])

    Cp = 128
    p1e_p = jnp.pad(p1e, ((0, 0), (0, 0), (0, Cp - C1h)))
    q1e_p = jnp.pad(q1e, ((0, 0), (0, 0), (0, Cp - C1h)))
    w2t_p = jnp.pad(s1['w2t'], ((0, Cp - C1h), (0, 0)))
    h1g = _sc_gather(p1e_p.reshape(B * M0, Cp), _flat_gather_idx(nidx1, M0), Cp)
    feat1 = pl.pallas_call(
        functools.partial(_mlp_max_body, kchunk=16),
        grid=(B,),
        in_specs=[pl.BlockSpec((_K * M1, Cp), lambda b: (b, 0)),
                  _batched((M1, Cp)), _full(w2t_p.shape)] +
                 [_full(s1[k].shape) for k in ['b2v', 'w3t', 'b3v']],
        out_specs=[_batched((M1, C1))],
        out_shape=[jax.ShapeDtypeStruct((B, M1, C1), f32)],
    )(h1g, q1e_p, w2t_p, *[s1[k] for k in ['b2v', 'w3t', 'b3v']])[0]

    s2_keys = ['wp2t', 'wf2t', 'q2w', 'b1v']
    nidx2, p2e, q2e, pos2 = pl.pallas_call(
        _sa2_sel_body,
        grid=(B,),
        in_specs=[_batched((M1, 3)), _batched((3, M1)), _batched((M1, C1)),
                  _batched((M2, 1))] +
                 [_full(s2[k].shape) for k in s2_keys],
        out_specs=[_batched((M2, _K)), _batched((M1, C2h)),
                   _batched((M2, C2h)), _batched((M2, 3))],
        out_shape=[jax.ShapeDtypeStruct((B, M2, _K), jnp.int32),
                   jax.ShapeDtypeStruct((B, M1, C2h), f32),
                   jax.ShapeDtypeStruct((B, M2, C2h), f32),
                   jax.ShapeDtypeStruct((B, M2, 3), f32)],
        scratch_shapes=[pltpu.VMEM((M2, M1), f32)],
    )(pos1, pos1t, feat1, i2c, *[s2[k] for k in s2_keys])

    h2g = _sc_gather(p2e.reshape(B * M1, C2h), _flat_gather_idx(nidx2, M1), C2h)
    feat2 = pl.pallas_call(
        functools.partial(_mlp_max_body, kchunk=_K),
        grid=(B,),
        in_specs=[pl.BlockSpec((_K * M2, C2h), lambda b: (b, 0)),
                  _batched((M2, C2h))] +
                 [_full(s2[k].shape) for k in ['w2t', 'b2v', 'w3t', 'b3v']],
        out_specs=[_batched((M2, C2))],
        out_shape=[jax.ShapeDtypeStruct((B, M2, C2), f32)],
    )(h2g, q2e, *[s2[k] for k in ['w2t', 'b2v', 'w3t', 'b3v']])[0]

    fin_keys = ['fp1', 'ff1', 'fb1', 'f2w', 'fb2', 'f3w', 'fb3']
    fc_keys = ['c1w', 'c1b', 'c2w', 'c2b', 'c3w', 'c3b']
    out = pl.pallas_call(
        _fin_body,
        grid=(B,),
        in_specs=[_batched((M2, 3)), _batched((M2, C2))] +
                 [_full(fin[k].shape) for k in fin_keys] +
                 [_full(fc[k].shape) for k in fc_keys],
        out_specs=[_batched((1, fc['c3w'].shape[1]))],
        out_shape=[jax.ShapeDtypeStruct((B, 1, fc['c3w'].shape[1]), f32)],
    )(pos2, feat2, *[fin[k] for k in fin_keys], *[fc[k] for k in fc_keys])[0]

    return out[:, 0, :]


# speedup vs baseline: 1.3515x; 1.3515x over previous
import functools

import jax
import jax.numpy as jnp
from jax import lax
from jax.experimental import pallas as pl
from jax.experimental.pallas import tpu as pltpu
from jax.experimental.pallas import tpu_sc as plsc

_K = 64
_INF = float('inf')

_NC, _NS = 2, 16
_NW = _NC * _NS
_CHUNK = 128
_NBUF = 4


def _dot(a, b, prec=None):
    return lax.dot_general(a, b, (((1,), (0,)), ((), ())),
                           precision=prec, preferred_element_type=jnp.float32)


def _split3(v):
    hi = v.astype(jnp.bfloat16)
    r1 = v - hi.astype(jnp.float32)
    mid = r1.astype(jnp.bfloat16)
    lo = (r1 - mid.astype(jnp.float32)).astype(jnp.bfloat16)
    return hi, mid, lo


def _gdot(oh, triple, data_left=False):
    ohb = oh.astype(jnp.bfloat16)
    if data_left:
        w = triple[0].shape[0]
        r = _dot(jnp.concatenate(triple, axis=0), ohb)
        return (r[:w] + r[w:2 * w]) + r[2 * w:]
    w = triple[0].shape[1]
    r = _dot(ohb, jnp.concatenate(triple, axis=1))
    return (r[:, :w] + r[:, w:2 * w]) + r[:, 2 * w:]


def _sqdist(q, pt):
    d0 = (q[:, 0:1] - pt[0:1, :]) ** 2
    d1 = (q[:, 1:2] - pt[1:2, :]) ** 2
    d2 = (q[:, 2:3] - pt[2:3, :]) ** 2
    return (d0 + d1) + d2


def _knn_select(D, d_ref):
    M, P = D.shape
    iot = lax.broadcasted_iota(jnp.int32, (M, P), 1)
    kio = lax.broadcasted_iota(jnp.int32, (M, _K), 1)
    big = jnp.int32(2 ** 30)
    d_ref[...] = D

    def step(k, nidx):
        D = d_ref[...]
        m = jnp.min(D, axis=1, keepdims=True)
        col = jnp.min(jnp.where(D == m, iot, big), axis=1, keepdims=True)
        d_ref[...] = jnp.where(iot == col, _INF, D)
        return jnp.where(kio == k, col, nidx)

    return lax.fori_loop(0, _K, step, jnp.zeros((M, _K), jnp.int32))


def _sa1_sel_body(ap_ref, apt_ref, i0c_ref, i0r_ref, i1c_ref, i1r_ref,
                  a1t_ref, q1w_ref, b1v_ref,
                  nidx_ref, p1e_ref, q1e_ref, pos1_ref, pos1t_ref, d_ref):
    ap = ap_ref[0]
    apt = apt_ref[0]
    i0c = i0c_ref[0]
    i0r = i0r_ref[0]
    i1c = i1c_ref[0]
    i1r = i1r_ref[0]
    N, M0, M1 = ap.shape[0], i0c.shape[0], i1c.shape[0]

    CH = min(N, 1024)
    ap3 = _split3(ap)
    apt3 = _split3(apt)
    pos0 = jnp.zeros((M0, 3), jnp.float32)
    pos0t = jnp.zeros((3, M0), jnp.float32)
    for s in range(0, N, CH):
        ii = lax.broadcasted_iota(jnp.int32, (M0, CH), 1) + s
        pos0 = pos0 + _gdot(i0c == ii, tuple(t[s:s + CH] for t in ap3))
        jj = lax.broadcasted_iota(jnp.int32, (CH, M0), 0) + s
        pos0t = pos0t + _gdot(jj == i0r, tuple(t[:, s:s + CH] for t in apt3),
                              data_left=True)

    oh1 = i1c == lax.broadcasted_iota(jnp.int32, (M1, M0), 1)
    q1 = _gdot(oh1, _split3(pos0))
    oh1t = lax.broadcasted_iota(jnp.int32, (M0, M1), 0) == i1r
    q1t = _gdot(oh1t, _split3(pos0t), data_left=True)

    p1e_ref[0] = _dot(pos0, a1t_ref[...], prec=None)
    q1e_ref[0] = _dot(q1, q1w_ref[...], prec=None) + b1v_ref[...]
    nidx_ref[0] = _knn_select(_sqdist(q1, pos0t), d_ref)
    pos1_ref[0] = q1
    pos1t_ref[0] = q1t


def _sa2_sel_body(p1_ref, p1t_ref, f1_ref, i2c_ref,
                  wp2t_ref, wf2t_ref, q2w_ref, b1v_ref,
                  nidx_ref, p2e_ref, q2e_ref, pos2_ref, d_ref):
    pos1 = p1_ref[0]
    pos1t = p1t_ref[0]
    feat1 = f1_ref[0]
    i2c = i2c_ref[0]
    M1, M2 = pos1.shape[0], i2c.shape[0]

    oh2 = i2c == lax.broadcasted_iota(jnp.int32, (M2, M1), 1)
    q2 = _gdot(oh2, _split3(pos1))
    p2e_ref[0] = (_dot(feat1, wf2t_ref[...], prec=None) +
                  _dot(pos1, wp2t_ref[...], prec=None))
    q2e_ref[0] = _dot(q2, q2w_ref[...], prec=None) + b1v_ref[...]
    nidx_ref[0] = _knn_select(_sqdist(q2, pos1t), d_ref)
    pos2_ref[0] = q2


def _mlp_max_body(h_ref, qe_ref, w2t_ref, b2v_ref, w3t_ref, b3v_ref, out_ref,
                  *, kchunk):
    KM, C1 = h_ref.shape
    M = qe_ref.shape[1]
    Kc = min(kchunk, KM // M)
    nch = (KM // M) // Kc
    qe = qe_ref[0]
    W2t, b2 = w2t_ref[...], b2v_ref[...]
    W3t, b3 = w3t_ref[...], b3v_ref[...]
    acc = jnp.full((M, W3t.shape[1]), -_INF, jnp.float32)
    for c in range(nch):
        blk = h_ref[pl.ds(c * Kc * M, Kc * M), :].reshape(Kc, M, C1)
        x = jnp.maximum(blk + qe, 0.0).reshape(Kc * M, C1)
        x = jnp.maximum(_dot(x, W2t, prec=None) + b2, 0.0)
        x = jnp.maximum(_dot(x, W3t, prec=None) + b3, 0.0)
        acc = jnp.maximum(acc, jnp.max(x.reshape(Kc, M, W3t.shape[1]), axis=0))
    out_ref[0] = acc


def _sc_gather(table, idx3d, C):
    cpw = idx3d.shape[1]
    rows_out = _NW * cpw * _CHUNK
    ngroups = cpw // _NBUF
    mesh = plsc.VectorSubcoreMesh(core_axis_name="c", subcore_axis_name="s")

    @functools.partial(
        pl.kernel, mesh=mesh,
        out_type=jax.ShapeDtypeStruct((rows_out, C), jnp.float32),
        scratch_types=[
            pltpu.VMEM((cpw, _CHUNK), jnp.int32),
            pltpu.VMEM((_NBUF, _CHUNK, C), jnp.float32),
            pltpu.SemaphoreType.DMA,
            pltpu.SemaphoreType.DMA,
        ],
    )
    def gk(table_hbm, idx_hbm, out_hbm, idx_v, rows_v, gsem, wsem):
        wid = lax.axis_index("s") * _NC + lax.axis_index("c")
        base = wid * cpw * _CHUNK
        pltpu.sync_copy(idx_hbm.at[wid], idx_v)

        def group(g, carry):
            row0 = base + g * (_NBUF * _CHUNK)
            gd, wd = [], []
            for j in range(_NBUF):
                gd.append(pltpu.async_copy(
                    table_hbm.at[idx_v.at[g * _NBUF + j]], rows_v.at[j], gsem))
            for j in range(_NBUF):
                gd[j].wait()
            for j in range(_NBUF):
                wd.append(pltpu.async_copy(
                    rows_v.at[j],
                    out_hbm.at[pl.ds(row0 + j * _CHUNK, _CHUNK)], wsem))
            for j in range(_NBUF):
                wd[j].wait()
            return carry

        lax.fori_loop(0, ngroups, group, 0)

    return gk(table, idx3d)


def _fin_body(p2_ref, f2_ref, fp1_ref, ff1_ref, fb1_ref, f2w_ref, fb2_ref,
              f3w_ref, fb3_ref, c1w_ref, c1b_ref, c2w_ref, c2b_ref,
              c3w_ref, c3b_ref, out_ref):
    pos2 = p2_ref[0]
    feat2 = f2_ref[0]
    x = _dot(pos2, fp1_ref[...], prec=None) + _dot(feat2, ff1_ref[...], prec=None)
    x = jnp.maximum(x + fb1_ref[...], 0.0)
    x = jnp.maximum(_dot(x, f2w_ref[...], prec=None) + fb2_ref[...], 0.0)
    x = jnp.maximum(_dot(x, f3w_ref[...], prec=None) + fb3_ref[...], 0.0)
    pooled = jnp.max(x, axis=0, keepdims=True)
    y = jnp.maximum(_dot(pooled, c1w_ref[...], prec=None) + c1b_ref[...], 0.0)
    y = jnp.maximum(_dot(y, c2w_ref[...], prec=None) + c2b_ref[...], 0.0)
    out_ref[0] = _dot(y, c3w_ref[...], prec=None) + c3b_ref[...]


def _full(shape):
    return pl.BlockSpec(shape, lambda b: (0,) * len(shape))


def _batched(shape):
    return pl.BlockSpec((1,) + shape, lambda b: (b,) + (0,) * len(shape))


def _flat_gather_idx(nidx, src_rows):
    B = nidx.shape[0]
    off = (jnp.arange(B, dtype=jnp.int32) * src_rows)[:, None, None]
    flat = (jnp.transpose(nidx, (0, 2, 1)) + off).reshape(-1)
    return flat.reshape(_NW, -1, _CHUNK)


def kernel(all_points, idx0, idx1, idx2, sa1_params, sa2_params, fin_params, fc_params):
    B, N, _ = all_points.shape
    M0, M1, M2 = idx0.shape[1], idx1.shape[1], idx2.shape[1]
    f32 = jnp.float32

    ap = all_points.astype(f32)
    apt = jnp.transpose(ap, (0, 2, 1))
    i0 = idx0.astype(jnp.int32)
    i1 = idx1.astype(jnp.int32)
    i2 = idx2.astype(jnp.int32)
    i0c, i0r = i0[:, :, None], i0[:, None, :]
    i1c, i1r = i1[:, :, None], i1[:, None, :]
    i2c = i2[:, :, None]

    (W1, g1, b1), (W2, g2, b2), (W3, g3, b3) = sa1_params
    a1t = ((W1[:, :3] + W1[:, 3:]) * g1[:, None]).T
    q1w = (-(W1[:, :3]) * g1[:, None]).T
    s1 = dict(a1t=a1t, q1w=q1w, b1v=b1[None, :],
              w2t=(W2 * g2[:, None]).T, b2v=b2[None, :],
              w3t=(W3 * g3[:, None]).T, b3v=b3[None, :])

    (V1, h1, c1), (V2, h2, c2), (V3, h3, c3) = sa2_params
    wp2t = (V1[:, :3] * h1[:, None]).T
    wf2t = (V1[:, 3:] * h1[:, None]).T
    s2 = dict(wp2t=wp2t, wf2t=wf2t, q2w=-wp2t, b1v=c1[None, :],
              w2t=(V2 * h2[:, None]).T, b2v=c2[None, :],
              w3t=(V3 * h3[:, None]).T, b3v=c3[None, :])

    (U1, e1, d1), (U2, e2, d2), (U3, e3, d3) = fin_params
    fin = dict(fp1=(U1[:, :3] * e1[:, None]).T, ff1=(U1[:, 3:] * e1[:, None]).T,
               fb1=d1[None, :],
               f2w=(U2 * e2[:, None]).T, fb2=d2[None, :],
               f3w=(U3 * e3[:, None]).T, fb3=d3[None, :])
    F1, fg1, fb1, F2, fg2, fb2, F3, fb3 = fc_params
    fc = dict(c1w=(F1 * fg1[:, None]).T, c1b=fb1[None, :],
              c2w=(F2 * fg2[:, None]).T, c2b=fb2[None, :],
              c3w=F3.T, c3b=fb3[None, :])

    C1h = s1['w2t'].shape[0]
    C1 = s1['w3t'].shape[1]
    C2h = s2['w2t'].shape[0]
    C2 = s2['w3t'].shape[1]

    s1_keys = ['a1t', 'q1w', 'b1v']
    nidx1, p1e, q1e, pos1, pos1t = pl.pallas_call(
        _sa1_sel_body,
        grid=(B,),
        in_specs=[_batched((N, 3)), _batched((3, N)),
                  _batched((M0, 1)), _batched((1, M0)),
                  _batched((M1, 1)), _batched((1, M1))] +
                 [_full(s1[k].shape) for k in s1_keys],
        out_specs=[_batched((M1, _K)), _batched((M0, C1h)),
                   _batched((M1, C1h)), _batched((M1, 3)), _batched((3, M1))],
        out_shape=[jax.ShapeDtypeStruct((B, M1, _K), jnp.int32),
                   jax.ShapeDtypeStruct((B, M0, C1h), f32),
                   jax.ShapeDtypeStruct((B, M1, C1h), f32),
                   jax.ShapeDtypeStruct((B, M1, 3), f32),
                   jax.ShapeDtypeStruct((B, 3, M1), f32)],
        scratch_shapes=[pltpu.VMEM((M1, M0), f32)],
    )(ap, apt, i0c, i0r, i1c, i1r, *[s1[k] for k in s1_keys])

    Cp = 128
    p1e_p = jnp.pad(p1e, ((0, 0), (0, 0), (0, Cp - C1h)))
    q1e_p = jnp.pad(q1e, ((0, 0), (0, 0), (0, Cp - C1h)))
    w2t_p = jnp.pad(s1['w2t'], ((0, Cp - C1h), (0, 0)))
    h1g = _sc_gather(p1e_p.reshape(B * M0, Cp), _flat_gather_idx(nidx1, M0), Cp)
    feat1 = pl.pallas_call(
        functools.partial(_mlp_max_body, kchunk=16),
        grid=(B,),
        in_specs=[pl.BlockSpec((_K * M1, Cp), lambda b: (b, 0)),
                  _batched((M1, Cp)), _full(w2t_p.shape)] +
                 [_full(s1[k].shape) for k in ['b2v', 'w3t', 'b3v']],
        out_specs=[_batched((M1, C1))],
        out_shape=[jax.ShapeDtypeStruct((B, M1, C1), f32)],
    )(h1g, q1e_p, w2t_p, *[s1[k] for k in ['b2v', 'w3t', 'b3v']])[0]

    s2_keys = ['wp2t', 'wf2t', 'q2w', 'b1v']
    nidx2, p2e, q2e, pos2 = pl.pallas_call(
        _sa2_sel_body,
        grid=(B,),
        in_specs=[_batched((M1, 3)), _batched((3, M1)), _batched((M1, C1)),
                  _batched((M2, 1))] +
                 [_full(s2[k].shape) for k in s2_keys],
        out_specs=[_batched((M2, _K)), _batched((M1, C2h)),
                   _batched((M2, C2h)), _batched((M2, 3))],
        out_shape=[jax.ShapeDtypeStruct((B, M2, _K), jnp.int32),
                   jax.ShapeDtypeStruct((B, M1, C2h), f32),
                   jax.ShapeDtypeStruct((B, M2, C2h), f32),
                   jax.ShapeDtypeStruct((B, M2, 3), f32)],
        scratch_shapes=[pltpu.VMEM((M2, M1), f32)],
    )(pos1, pos1t, feat1, i2c, *[s2[k] for k in s2_keys])

    h2g = _sc_gather(p2e.reshape(B * M1, C2h), _flat_gather_idx(nidx2, M1), C2h)
    feat2 = pl.pallas_call(
        functools.partial(_mlp_max_body, kchunk=_K),
        grid=(B,),
        in_specs=[pl.BlockSpec((_K * M2, C2h), lambda b: (b, 0)),
                  _batched((M2, C2h))] +
                 [_full(s2[k].shape) for k in ['w2t', 'b2v', 'w3t', 'b3v']],
        out_specs=[_batched((M2, C2))],
        out_shape=[jax.ShapeDtypeStruct((B, M2, C2), f32)],
    )(h2g, q2e, *[s2[k] for k in ['w2t', 'b2v', 'w3t', 'b3v']])[0]

    fin_keys = ['fp1', 'ff1', 'fb1', 'f2w', 'fb2', 'f3w', 'fb3']
    fc_keys = ['c1w', 'c1b', 'c2w', 'c2b', 'c3w', 'c3b']
    out = pl.pallas_call(
        _fin_body,
        grid=(B,),
        in_specs=[_batched((M2, 3)), _batched((M2, C2))] +
                 [_full(fin[k].shape) for k in fin_keys] +
                 [_full(fc[k].shape) for k in fc_keys],
        out_specs=[_batched((1, fc['c3w'].shape[1]))],
        out_shape=[jax.ShapeDtypeStruct((B, 1, fc['c3w'].shape[1]), f32)],
    )(pos2, feat2, *[fin[k] for k in fin_keys], *[fc[k] for k in fc_keys])[0]

    return out[:, 0, :]
